# baseline (speedup 1.0000x reference)
import numpy as np
import jax
import jax.numpy as jnp
from jax import lax
from jax.experimental import pallas as pl
from jax.experimental.pallas import tpu as pltpu
from jax.experimental.pallas import tpu_sc as plsc

N = 10000
N0 = 10000
H = 32
DIM = 3
IN_FDIM = 128
OUT_FDIM = 128
K = 15
EXTENT = 1.0 * 2.5 / 5.0
NEG_SLOPE = 0.1

NE = N * H
CPAD = 16

BN = 80
E = BN * H

_SC_INFO = plsc.get_sparse_core_info()
NW = _SC_INFO.num_cores * _SC_INFO.num_subcores
EW = NE // NW
C = 80
NB = 5
ITERS = EW // C
GROUPS = ITERS // NB


def _kernel_points_np():
    rng = np.random.RandomState(42)
    dirs = rng.normal(size=(K - 1, DIM))
    dirs = dirs / (np.linalg.norm(dirs, axis=1, keepdims=True) + 1e-9)
    radii = rng.uniform(size=(K - 1, 1)) ** (1.0 / 3.0) * EXTENT
    return np.concatenate([np.zeros((1, DIM)), dirs * radii], axis=0).astype(np.float32)


_KP = _kernel_points_np()
_KP_T = np.ascontiguousarray(_KP.T)
_KP_SQ = np.sum(_KP * _KP, axis=1, keepdims=True).T.copy()


def _sc_gather(feat_hbm, spp_hbm, idx_hbm, xg_hbm, dn_hbm,
               idx_all, fbufs, cbufs, gsems, ssems):
    wid = lax.axis_index("s") * _SC_INFO.num_cores + lax.axis_index("c")
    ebase = wid * EW

    pltpu.sync_copy(idx_hbm.at[pl.ds(ebase, EW)], idx_all)

    def fire_gather(b, it):
        off = pl.multiple_of(it * C, 8)
        idx_v = idx_all.at[pl.ds(off, C)]
        pltpu.async_copy(feat_hbm.at[idx_v], fbufs.at[b], gsems.at[b])
        pltpu.async_copy(spp_hbm.at[idx_v], cbufs.at[b], gsems.at[b])

    def drain_gather(b):
        pltpu.make_async_copy(feat_hbm.at[pl.ds(0, C)], fbufs.at[b],
                              gsems.at[b]).wait()
        pltpu.make_async_copy(spp_hbm.at[pl.ds(0, C)], cbufs.at[b],
                              gsems.at[b]).wait()

    def fire_scatter(b, it):
        row = ebase + it * C
        pltpu.async_copy(fbufs.at[b], xg_hbm.at[pl.ds(row, C)], ssems.at[b])
        pltpu.async_copy(cbufs.at[b], dn_hbm.at[pl.ds(row, C)], ssems.at[b])

    def drain_scatter(b):
        pltpu.make_async_copy(fbufs.at[b], xg_hbm.at[pl.ds(0, C)],
                              ssems.at[b]).wait()
        pltpu.make_async_copy(cbufs.at[b], dn_hbm.at[pl.ds(0, C)],
                              ssems.at[b]).wait()

    for b in range(NB):
        fire_gather(b, b)

    def group(g, carry):
        for b in range(NB):
            drain_gather(b)
            fire_scatter(b, g * NB + b)
        for b in range(NB):
            drain_scatter(b)

            @pl.when(g < GROUPS - 1)
            def _():
                fire_gather(b, (g + 1) * NB + b)
        return carry

    lax.fori_loop(0, GROUPS, group, 0)


def _gather_stage(features, spp, idx_flat):
    mesh = plsc.VectorSubcoreMesh(core_axis_name="c", subcore_axis_name="s")
    f = pl.kernel(
        _sc_gather,
        mesh=mesh,
        out_type=[
            jax.ShapeDtypeStruct((NE, IN_FDIM // 2), jnp.int32),
            jax.ShapeDtypeStruct((NE, CPAD), jnp.float32),
        ],
        scratch_types=[
            pltpu.VMEM((EW,), jnp.int32),
            pltpu.VMEM((NB, C, IN_FDIM // 2), jnp.int32),
            pltpu.VMEM((NB, C, CPAD), jnp.float32),
            pltpu.SemaphoreType.DMA((NB,)),
            pltpu.SemaphoreType.DMA((NB,)),
        ],
        compiler_params=pltpu.CompilerParams(use_tc_tiling_on_sc=False),
    )
    return f(features, spp, idx_flat)


_M8_NP = np.concatenate([
    np.ones((3, K), np.float32),
    np.zeros((1, K), np.float32),
    (-2.0 * _KP_T).astype(np.float32),
    _KP_SQ.astype(np.float32),
], axis=0)
_INV_EXTENT = float(1.0 / EXTENT)


def _tc_body(qprep_ref, xg_ref, dn_ref, kve_ref, kvo_ref, m8_ref, out_ref):
    diff4 = dn_ref[:, 0:4] - qprep_ref[:, :]
    diff8 = jnp.concatenate([diff4 * diff4, diff4], axis=1)
    sq = jnp.dot(diff8, m8_ref[:, :],
                 preferred_element_type=jnp.float32)
    dist = jnp.sqrt(jnp.maximum(sq, 1e-12))
    w = jnp.maximum(1.0 - dist * _INV_EXTENT, 0.0)

    xi = xg_ref[:, :]
    xlo = lax.bitcast_convert_type(xi << 16, jnp.float32)
    xhi = lax.bitcast_convert_type(xi & jnp.int32(-65536), jnp.float32)

    w3 = w.reshape(BN, H, K)
    dnums = (((1,), (1,)), ((0,), (0,)))
    we = lax.dot_general(w3, xlo.reshape(BN, H, IN_FDIM // 2), dnums,
                         preferred_element_type=jnp.float32)
    wo = lax.dot_general(w3, xhi.reshape(BN, H, IN_FDIM // 2), dnums,
                         preferred_element_type=jnp.float32)
    acc = jnp.zeros((BN, OUT_FDIM), jnp.float32)
    for k in range(K):
        acc = acc + jnp.dot(we[:, k, :], kve_ref[k],
                            preferred_element_type=jnp.float32)
        acc = acc + jnp.dot(wo[:, k, :], kvo_ref[k],
                            preferred_element_type=jnp.float32)
    out_ref[:, :] = jnp.where(acc >= 0, acc, NEG_SLOPE * acc)


def kernel(query_points, support_points, neighbors_indices, features, K_values):
    spp = jnp.pad(support_points, ((0, 0), (0, CPAD - DIM)))
    idx_flat = neighbors_indices.reshape(-1)
    qp4 = jnp.concatenate(
        [query_points, jnp.full((N, 1), -1.0, jnp.float32)], axis=1)
    qprep = jnp.repeat(qp4, H, axis=0)

    fb = lax.bitcast_convert_type(
        features.astype(jnp.bfloat16), jnp.uint16).astype(jnp.uint32)
    packed = lax.bitcast_convert_type(
        fb[:, 0::2] | (fb[:, 1::2] << 16), jnp.int32)
    xg, dn = _gather_stage(packed, spp, idx_flat)

    out = pl.pallas_call(
        _tc_body,
        grid=(N // BN,),
        in_specs=[
            pl.BlockSpec((E, 4), lambda i: (i, 0)),
            pl.BlockSpec((E, IN_FDIM // 2), lambda i: (i, 0)),
            pl.BlockSpec((E, CPAD), lambda i: (i, 0)),
            pl.BlockSpec((K, IN_FDIM // 2, OUT_FDIM), lambda i: (0, 0, 0)),
            pl.BlockSpec((K, IN_FDIM // 2, OUT_FDIM), lambda i: (0, 0, 0)),
            pl.BlockSpec((8, K), lambda i: (0, 0)),
        ],
        out_specs=pl.BlockSpec((BN, OUT_FDIM), lambda i: (i, 0)),
        out_shape=jax.ShapeDtypeStruct((N, OUT_FDIM), jnp.float32),
    )(qprep, xg, dn, K_values[:, 0::2, :], K_values[:, 1::2, :],
      jnp.asarray(_M8_NP))
    return out


# speedup vs baseline: 1.5333x; 1.5333x over previous
import numpy as np
import jax
import jax.numpy as jnp
from jax import lax
from jax.experimental import pallas as pl
from jax.experimental.pallas import tpu as pltpu
from jax.experimental.pallas import tpu_sc as plsc

N = 10000
N0 = 10000
H = 32
DIM = 3
IN_FDIM = 128
OUT_FDIM = 128
K = 15
EXTENT = 1.0 * 2.5 / 5.0
NEG_SLOPE = 0.1

NE = N * H
CPAD = 16

BN = 80
E = BN * H

_SC_INFO = plsc.get_sparse_core_info()
NW = _SC_INFO.num_cores * _SC_INFO.num_subcores
EW = NE // NW
C = 80
NB = 5
ITERS = EW // C
GROUPS = ITERS // NB


def _kernel_points_np():
    rng = np.random.RandomState(42)
    dirs = rng.normal(size=(K - 1, DIM))
    dirs = dirs / (np.linalg.norm(dirs, axis=1, keepdims=True) + 1e-9)
    radii = rng.uniform(size=(K - 1, 1)) ** (1.0 / 3.0) * EXTENT
    return np.concatenate([np.zeros((1, DIM)), dirs * radii], axis=0).astype(np.float32)


_KP = _kernel_points_np()
_KP_T = np.ascontiguousarray(_KP.T)
_KP_SQ = np.sum(_KP * _KP, axis=1, keepdims=True).T.copy()


def _sc_gather(feat_hbm, spp_hbm, idx_hbm, xg_hbm, dn_hbm,
               idx_all, fbufs, cbufs, gsems, ssems):
    wid = lax.axis_index("s") * _SC_INFO.num_cores + lax.axis_index("c")
    ebase = wid * EW

    pltpu.sync_copy(idx_hbm.at[pl.ds(ebase, EW)], idx_all)

    def fire_gather(b, it):
        off = pl.multiple_of(it * C, 8)
        idx_v = idx_all.at[pl.ds(off, C)]
        pltpu.async_copy(feat_hbm.at[idx_v], fbufs.at[b], gsems.at[b])
        pltpu.async_copy(spp_hbm.at[idx_v], cbufs.at[b], gsems.at[b])

    def drain_gather(b):
        pltpu.make_async_copy(feat_hbm.at[pl.ds(0, C)], fbufs.at[b],
                              gsems.at[b]).wait()
        pltpu.make_async_copy(spp_hbm.at[pl.ds(0, C)], cbufs.at[b],
                              gsems.at[b]).wait()

    def fire_scatter(b, it):
        row = ebase + it * C
        pltpu.async_copy(fbufs.at[b], xg_hbm.at[pl.ds(row, C)], ssems.at[b])
        pltpu.async_copy(cbufs.at[b], dn_hbm.at[pl.ds(row, C)], ssems.at[b])

    def drain_scatter(b):
        pltpu.make_async_copy(fbufs.at[b], xg_hbm.at[pl.ds(0, C)],
                              ssems.at[b]).wait()
        pltpu.make_async_copy(cbufs.at[b], dn_hbm.at[pl.ds(0, C)],
                              ssems.at[b]).wait()

    for b in range(NB):
        fire_gather(b, b)

    def group(g, carry):
        for b in range(NB):
            it = g * NB + b
            drain_gather(b)
            fire_scatter(b, it)
            drain_scatter(b)

            @pl.when(it + NB < ITERS)
            def _():
                fire_gather(b, it + NB)
        return carry

    lax.fori_loop(0, GROUPS, group, 0)


def _gather_stage(features, spp, idx_flat):
    mesh = plsc.VectorSubcoreMesh(core_axis_name="c", subcore_axis_name="s")
    f = pl.kernel(
        _sc_gather,
        mesh=mesh,
        out_type=[
            jax.ShapeDtypeStruct((NE, IN_FDIM), jnp.float32),
            jax.ShapeDtypeStruct((NE, CPAD), jnp.float32),
        ],
        scratch_types=[
            pltpu.VMEM((EW,), jnp.int32),
            pltpu.VMEM((NB, C, IN_FDIM), jnp.float32),
            pltpu.VMEM((NB, C, CPAD), jnp.float32),
            pltpu.SemaphoreType.DMA((NB,)),
            pltpu.SemaphoreType.DMA((NB,)),
        ],
        compiler_params=pltpu.CompilerParams(use_tc_tiling_on_sc=False),
    )
    return f(features, spp, idx_flat)


_M8_NP = np.concatenate([
    np.ones((3, K), np.float32),
    np.zeros((1, K), np.float32),
    (-2.0 * _KP_T).astype(np.float32),
    _KP_SQ.astype(np.float32),
], axis=0)
_INV_EXTENT = float(1.0 / EXTENT)


def _tc_body(qprep_ref, xg_ref, dn_ref, kv_ref, m8_ref, out_ref):
    diff4 = dn_ref[:, 0:4] - qprep_ref[:, :]
    diff8 = jnp.concatenate([diff4 * diff4, diff4], axis=1)
    sq = jnp.dot(diff8, m8_ref[:, :],
                 preferred_element_type=jnp.float32)
    dist = jnp.sqrt(jnp.maximum(sq, 1e-12))
    w = jnp.maximum(1.0 - dist * _INV_EXTENT, 0.0)

    w3 = w.reshape(BN, H, K)
    xg3 = xg_ref[:, :].reshape(BN, H, IN_FDIM)
    weighted = lax.dot_general(
        w3, xg3, (((1,), (1,)), ((0,), (0,))),
        preferred_element_type=jnp.float32)
    acc = jnp.zeros((BN, OUT_FDIM), jnp.float32)
    for k in range(K):
        acc = acc + jnp.dot(weighted[:, k, :], kv_ref[k],
                            preferred_element_type=jnp.float32)
    out_ref[:, :] = jnp.where(acc >= 0, acc, NEG_SLOPE * acc)


def kernel(query_points, support_points, neighbors_indices, features, K_values):
    spp = jnp.pad(support_points, ((0, 0), (0, CPAD - DIM)))
    idx_flat = neighbors_indices.reshape(-1)
    qp4 = jnp.concatenate(
        [query_points, jnp.full((N, 1), -1.0, jnp.float32)], axis=1)
    qprep = jnp.repeat(qp4, H, axis=0)

    xg, dn = _gather_stage(features, spp, idx_flat)

    out = pl.pallas_call(
        _tc_body,
        grid=(N // BN,),
        in_specs=[
            pl.BlockSpec((E, 4), lambda i: (i, 0)),
            pl.BlockSpec((E, IN_FDIM), lambda i: (i, 0)),
            pl.BlockSpec((E, CPAD), lambda i: (i, 0)),
            pl.BlockSpec((K, IN_FDIM, OUT_FDIM), lambda i: (0, 0, 0)),
            pl.BlockSpec((8, K), lambda i: (0, 0)),
        ],
        out_specs=pl.BlockSpec((BN, OUT_FDIM), lambda i: (i, 0)),
        out_shape=jax.ShapeDtypeStruct((N, OUT_FDIM), jnp.float32),
    )(qprep, xg, dn, K_values, jnp.asarray(_M8_NP))
    return out


# speedup vs baseline: 1.6585x; 1.0817x over previous
import numpy as np
import jax
import jax.numpy as jnp
from jax import lax
from jax.experimental import pallas as pl
from jax.experimental.pallas import tpu as pltpu
from jax.experimental.pallas import tpu_sc as plsc

N = 10000
N0 = 10000
H = 32
DIM = 3
IN_FDIM = 128
OUT_FDIM = 128
K = 15
EXTENT = 1.0 * 2.5 / 5.0
NEG_SLOPE = 0.1

NE = N * H
CPAD = 16

BN = 200
E = BN * H

_SC_INFO = plsc.get_sparse_core_info()
NW = _SC_INFO.num_cores * _SC_INFO.num_subcores
EW = NE // NW
C = 80
NB = 5
ITERS = EW // C
GROUPS = ITERS // NB


def _kernel_points_np():
    rng = np.random.RandomState(42)
    dirs = rng.normal(size=(K - 1, DIM))
    dirs = dirs / (np.linalg.norm(dirs, axis=1, keepdims=True) + 1e-9)
    radii = rng.uniform(size=(K - 1, 1)) ** (1.0 / 3.0) * EXTENT
    return np.concatenate([np.zeros((1, DIM)), dirs * radii], axis=0).astype(np.float32)


_KP = _kernel_points_np()
_KP_T = np.ascontiguousarray(_KP.T)
_KP_SQ = np.sum(_KP * _KP, axis=1, keepdims=True).T.copy()


def _sc_gather(feat_hbm, spp_hbm, idx_hbm, xg_hbm, dn_hbm,
               idx_all, fbufs, cbufs, gsems, ssems):
    wid = lax.axis_index("s") * _SC_INFO.num_cores + lax.axis_index("c")
    ebase = wid * EW

    pltpu.sync_copy(idx_hbm.at[pl.ds(ebase, EW)], idx_all)

    def fire_gather(b, it):
        off = pl.multiple_of(it * C, 8)
        idx_v = idx_all.at[pl.ds(off, C)]
        pltpu.async_copy(feat_hbm.at[idx_v], fbufs.at[b], gsems.at[b])
        pltpu.async_copy(spp_hbm.at[idx_v], cbufs.at[b], gsems.at[b])

    def drain_gather(b):
        pltpu.make_async_copy(feat_hbm.at[pl.ds(0, C)], fbufs.at[b],
                              gsems.at[b]).wait()
        pltpu.make_async_copy(spp_hbm.at[pl.ds(0, C)], cbufs.at[b],
                              gsems.at[b]).wait()

    def fire_scatter(b, it):
        row = ebase + it * C
        pltpu.async_copy(fbufs.at[b], xg_hbm.at[pl.ds(row, C)], ssems.at[b])
        pltpu.async_copy(cbufs.at[b], dn_hbm.at[pl.ds(row, C)], ssems.at[b])

    def drain_scatter(b):
        pltpu.make_async_copy(fbufs.at[b], xg_hbm.at[pl.ds(0, C)],
                              ssems.at[b]).wait()
        pltpu.make_async_copy(cbufs.at[b], dn_hbm.at[pl.ds(0, C)],
                              ssems.at[b]).wait()

    for b in range(NB):
        fire_gather(b, b)

    def group(g, carry):
        for b in range(NB):
            it = g * NB + b
            drain_gather(b)
            fire_scatter(b, it)
            drain_scatter(b)

            @pl.when(it + NB < ITERS)
            def _():
                fire_gather(b, it + NB)
        return carry

    lax.fori_loop(0, GROUPS, group, 0)


def _gather_stage(features, spp, idx_flat):
    mesh = plsc.VectorSubcoreMesh(core_axis_name="c", subcore_axis_name="s")
    f = pl.kernel(
        _sc_gather,
        mesh=mesh,
        out_type=[
            jax.ShapeDtypeStruct((NE, IN_FDIM), jnp.float32),
            jax.ShapeDtypeStruct((NE, CPAD), jnp.float32),
        ],
        scratch_types=[
            pltpu.VMEM((EW,), jnp.int32),
            pltpu.VMEM((NB, C, IN_FDIM), jnp.float32),
            pltpu.VMEM((NB, C, CPAD), jnp.float32),
            pltpu.SemaphoreType.DMA((NB,)),
            pltpu.SemaphoreType.DMA((NB,)),
        ],
        compiler_params=pltpu.CompilerParams(use_tc_tiling_on_sc=False),
    )
    return f(features, spp, idx_flat)


_M8_NP = np.concatenate([
    np.ones((3, K), np.float32),
    np.zeros((1, K), np.float32),
    (-2.0 * _KP_T).astype(np.float32),
    _KP_SQ.astype(np.float32),
], axis=0)
_INV_EXTENT = float(1.0 / EXTENT)


def _tc_body(qprep_ref, xg_ref, dn_ref, kv_ref, m8_ref, out_ref):
    diff4 = dn_ref[:, 0:4] - qprep_ref[:, :]
    diff8 = jnp.concatenate([diff4 * diff4, diff4], axis=1)
    sq = jnp.dot(diff8, m8_ref[:, :],
                 preferred_element_type=jnp.float32)
    dist = jnp.sqrt(jnp.maximum(sq, 1e-12))
    w = jnp.maximum(1.0 - dist * _INV_EXTENT, 0.0)

    w3 = w.reshape(BN, H, K)
    xg3 = xg_ref[:, :].reshape(BN, H, IN_FDIM)
    weighted = lax.dot_general(
        w3, xg3, (((1,), (1,)), ((0,), (0,))),
        preferred_element_type=jnp.float32)
    acc = jnp.zeros((BN, OUT_FDIM), jnp.float32)
    for k in range(K):
        acc = acc + jnp.dot(weighted[:, k, :], kv_ref[k],
                            preferred_element_type=jnp.float32)
    out_ref[:, :] = jnp.where(acc >= 0, acc, NEG_SLOPE * acc)


def kernel(query_points, support_points, neighbors_indices, features, K_values):
    spp = jnp.pad(support_points, ((0, 0), (0, CPAD - DIM)))
    idx_flat = neighbors_indices.reshape(-1)
    qp4 = jnp.concatenate(
        [query_points, jnp.full((N, 1), -1.0, jnp.float32)], axis=1)
    qprep = jnp.repeat(qp4, H, axis=0)

    xg, dn = _gather_stage(features, spp, idx_flat)

    out = pl.pallas_call(
        _tc_body,
        grid=(N // BN,),
        in_specs=[
            pl.BlockSpec((E, 4), lambda i: (i, 0)),
            pl.BlockSpec((E, IN_FDIM), lambda i: (i, 0)),
            pl.BlockSpec((E, CPAD), lambda i: (i, 0)),
            pl.BlockSpec((K, IN_FDIM, OUT_FDIM), lambda i: (0, 0, 0)),
            pl.BlockSpec((8, K), lambda i: (0, 0)),
        ],
        out_specs=pl.BlockSpec((BN, OUT_FDIM), lambda i: (i, 0)),
        out_shape=jax.ShapeDtypeStruct((N, OUT_FDIM), jnp.float32),
    )(qprep, xg, dn, K_values, jnp.asarray(_M8_NP))
    return out


# speedup vs baseline: 1.7414x; 1.0500x over previous
import numpy as np
import jax
import jax.numpy as jnp
from jax import lax
from jax.experimental import pallas as pl
from jax.experimental.pallas import tpu as pltpu
from jax.experimental.pallas import tpu_sc as plsc

N = 10000
N0 = 10000
H = 32
DIM = 3
IN_FDIM = 128
OUT_FDIM = 128
K = 15
EXTENT = 1.0 * 2.5 / 5.0
NEG_SLOPE = 0.1

NE = N * H
CPAD = 16

BN = 400
E = BN * H

_SC_INFO = plsc.get_sparse_core_info()
NW = _SC_INFO.num_cores * _SC_INFO.num_subcores
EW = NE // NW
C = 80
NB = 5
ITERS = EW // C
GROUPS = ITERS // NB


def _kernel_points_np():
    rng = np.random.RandomState(42)
    dirs = rng.normal(size=(K - 1, DIM))
    dirs = dirs / (np.linalg.norm(dirs, axis=1, keepdims=True) + 1e-9)
    radii = rng.uniform(size=(K - 1, 1)) ** (1.0 / 3.0) * EXTENT
    return np.concatenate([np.zeros((1, DIM)), dirs * radii], axis=0).astype(np.float32)


_KP = _kernel_points_np()
_KP_T = np.ascontiguousarray(_KP.T)
_KP_SQ = np.sum(_KP * _KP, axis=1, keepdims=True).T.copy()


def _sc_gather(feat_hbm, spp_hbm, idx_hbm, xg_hbm, dn_hbm,
               idx_all, fbufs, cbufs, gsems, ssems):
    wid = lax.axis_index("s") * _SC_INFO.num_cores + lax.axis_index("c")
    ebase = wid * EW

    pltpu.sync_copy(idx_hbm.at[pl.ds(ebase, EW)], idx_all)

    def fire_gather(b, it):
        off = pl.multiple_of(it * C, 8)
        idx_v = idx_all.at[pl.ds(off, C)]
        pltpu.async_copy(feat_hbm.at[idx_v], fbufs.at[b], gsems.at[b])
        pltpu.async_copy(spp_hbm.at[idx_v], cbufs.at[b], gsems.at[b])

    def drain_gather(b):
        pltpu.make_async_copy(feat_hbm.at[pl.ds(0, C)], fbufs.at[b],
                              gsems.at[b]).wait()
        pltpu.make_async_copy(spp_hbm.at[pl.ds(0, C)], cbufs.at[b],
                              gsems.at[b]).wait()

    def fire_scatter(b, it):
        row = ebase + it * C
        pltpu.async_copy(fbufs.at[b], xg_hbm.at[pl.ds(row, C)], ssems.at[b])
        pltpu.async_copy(cbufs.at[b], dn_hbm.at[pl.ds(row, C)], ssems.at[b])

    def drain_scatter(b):
        pltpu.make_async_copy(fbufs.at[b], xg_hbm.at[pl.ds(0, C)],
                              ssems.at[b]).wait()
        pltpu.make_async_copy(cbufs.at[b], dn_hbm.at[pl.ds(0, C)],
                              ssems.at[b]).wait()

    for b in range(NB):
        fire_gather(b, b)

    def group(g, carry):
        for b in range(NB):
            it = g * NB + b
            drain_gather(b)
            fire_scatter(b, it)
            drain_scatter(b)

            @pl.when(it + NB < ITERS)
            def _():
                fire_gather(b, it + NB)
        return carry

    lax.fori_loop(0, GROUPS, group, 0)


def _gather_stage(features, spp, idx_flat):
    mesh = plsc.VectorSubcoreMesh(core_axis_name="c", subcore_axis_name="s")
    f = pl.kernel(
        _sc_gather,
        mesh=mesh,
        out_type=[
            jax.ShapeDtypeStruct((NE, IN_FDIM), jnp.float32),
            jax.ShapeDtypeStruct((NE, CPAD), jnp.float32),
        ],
        scratch_types=[
            pltpu.VMEM((EW,), jnp.int32),
            pltpu.VMEM((NB, C, IN_FDIM), jnp.float32),
            pltpu.VMEM((NB, C, CPAD), jnp.float32),
            pltpu.SemaphoreType.DMA((NB,)),
            pltpu.SemaphoreType.DMA((NB,)),
        ],
        compiler_params=pltpu.CompilerParams(use_tc_tiling_on_sc=False),
    )
    return f(features, spp, idx_flat)


_M8_NP = np.concatenate([
    np.ones((3, K), np.float32),
    np.zeros((1, K), np.float32),
    (-2.0 * _KP_T).astype(np.float32),
    _KP_SQ.astype(np.float32),
], axis=0)
_INV_EXTENT = float(1.0 / EXTENT)


def _tc_body(qprep_ref, xg_ref, dn_ref, kv_ref, m8_ref, out_ref):
    diff4 = dn_ref[:, 0:4] - qprep_ref[:, :]
    diff8 = jnp.concatenate([diff4 * diff4, diff4], axis=1)
    sq = jnp.dot(diff8, m8_ref[:, :],
                 preferred_element_type=jnp.float32)
    dist = jnp.sqrt(jnp.maximum(sq, 1e-12))
    w = jnp.maximum(1.0 - dist * _INV_EXTENT, 0.0)

    w3 = w.reshape(BN, H, K)
    xg3 = xg_ref[:, :].reshape(BN, H, IN_FDIM)
    weighted = lax.dot_general(
        w3, xg3, (((1,), (1,)), ((0,), (0,))),
        preferred_element_type=jnp.float32)
    acc = jnp.zeros((BN, OUT_FDIM), jnp.float32)
    for k in range(K):
        acc = acc + jnp.dot(weighted[:, k, :], kv_ref[k],
                            preferred_element_type=jnp.float32)
    out_ref[:, :] = jnp.where(acc >= 0, acc, NEG_SLOPE * acc)


def kernel(query_points, support_points, neighbors_indices, features, K_values):
    spp = jnp.pad(support_points, ((0, 0), (0, CPAD - DIM)))
    idx_flat = neighbors_indices.reshape(-1)
    qp4 = jnp.concatenate(
        [query_points, jnp.full((N, 1), -1.0, jnp.float32)], axis=1)
    qprep = jnp.repeat(qp4, H, axis=0)

    xg, dn = _gather_stage(features, spp, idx_flat)

    out = pl.pallas_call(
        _tc_body,
        grid=(N // BN,),
        in_specs=[
            pl.BlockSpec((E, 4), lambda i: (i, 0)),
            pl.BlockSpec((E, IN_FDIM), lambda i: (i, 0)),
            pl.BlockSpec((E, CPAD), lambda i: (i, 0)),
            pl.BlockSpec((K, IN_FDIM, OUT_FDIM), lambda i: (0, 0, 0)),
            pl.BlockSpec((8, K), lambda i: (0, 0)),
        ],
        out_specs=pl.BlockSpec((BN, OUT_FDIM), lambda i: (i, 0)),
        out_shape=jax.ShapeDtypeStruct((N, OUT_FDIM), jnp.float32),
    )(qprep, xg, dn, K_values, jnp.asarray(_M8_NP))
    return out
